# baseline (speedup 1.0000x reference)
import functools

import jax
import jax.numpy as jnp
from jax import lax
from jax.experimental import pallas as pl
from jax.experimental.pallas import tpu as pltpu
from jax.experimental.pallas import tpu_sc as plsc

_TB = 512
_CDIM = 64

_N = 4 * 64 * 64 * 2048
_NW = 32
_CHUNK = 65536


def _sc_zeros():
    mesh = plsc.VectorSubcoreMesh(core_axis_name="c", subcore_axis_name="s")
    n_w = _N // _NW

    @functools.partial(
        pl.kernel,
        out_type=jax.ShapeDtypeStruct((_N,), jnp.float32),
        mesh=mesh,
        scratch_types=[
            pltpu.VMEM((_CHUNK,), jnp.float32),
            pltpu.SemaphoreType.DMA,
        ],
    )
    def k(out_hbm, zbuf, sem):
        wid = lax.axis_index("s") * 2 + lax.axis_index("c")

        def zero_body(i, carry):
            zbuf[pl.ds(i * 16, 16)] = jnp.zeros((16,), jnp.float32)
            return carry

        lax.fori_loop(0, _CHUNK // 16, zero_body, 0)
        base = wid * n_w
        for i in range(n_w // _CHUNK):
            pltpu.async_copy(
                zbuf, out_hbm.at[pl.ds(base + i * _CHUNK, _CHUNK)], sem
            ).wait()

    return k()


def _router_block(x_ref, w_ref, b_ref, ne_ref, cap_ref, out_ref, carry_ref):
    tb = pl.program_id(1)
    e_dim = w_ref.shape[1]

    x = x_ref[0].astype(jnp.bfloat16)
    w = w_ref[...].astype(jnp.bfloat16)
    acc = jnp.dot(x, w, preferred_element_type=jnp.float32)
    acc_t = acc.T
    logits = (acc_t.astype(jnp.bfloat16) + b_ref[...].astype(jnp.bfloat16))
    logits = logits.astype(jnp.float32)

    lmax = jnp.max(logits, axis=0, keepdims=True)
    ex = jnp.exp(logits - lmax)
    ssum = jnp.sum(ex, axis=0, keepdims=True)
    probs = ex / ssum
    pmax = jnp.max(probs, axis=0, keepdims=True)
    eiota = jax.lax.broadcasted_iota(jnp.int32, (e_dim, _TB), 0)
    idx = jnp.min(jnp.where(probs == pmax, eiota, e_dim), axis=0, keepdims=True)

    ne = ne_ref[0, 0]
    mask = ((eiota == idx) & (eiota < ne)).astype(jnp.float32)

    ui = jax.lax.broadcasted_iota(jnp.int32, (_TB, _TB), 0)
    uj = jax.lax.broadcasted_iota(jnp.int32, (_TB, _TB), 1)
    triu = (ui <= uj).astype(jnp.float32)
    cs = jnp.dot(mask, triu, preferred_element_type=jnp.float32)

    @pl.when(tb == 0)
    def _():
        carry_ref[...] = jnp.zeros_like(carry_ref)

    carry = carry_ref[:, 0:1]
    pie = (cs + carry) * mask
    carry_ref[:, 0:1] = carry + cs[:, _TB - 1 : _TB]

    pos = jnp.sum(pie, axis=0, keepdims=True)
    cap = cap_ref[0, 0].astype(jnp.float32)
    wc = (pos > 0.0) & (pos <= cap)
    gate = jnp.where(wc, pmax, 0.0)
    c0 = jnp.where(wc, pos - 1.0, 0.0).astype(jnp.int32)

    gm = mask * gate
    ciota = jax.lax.broadcasted_iota(jnp.int32, (_CDIM, _TB), 0)
    cm = (ciota == c0).astype(jnp.float32)
    out_ref[0] = gm[:, None, :] * cm[None, :, :]


def kernel(token_inputs, W, b, num_experts, expert_capacity):
    g_dim, t_dim, d_dim = token_inputs.shape
    e_dim = W.shape[1]
    nt = t_dim // _TB

    b2 = b.reshape(e_dim, 1)
    ne = jnp.asarray(num_experts, jnp.int32).reshape(1, 1)
    cap = jnp.asarray(expert_capacity, jnp.int32).reshape(1, 1)

    out = pl.pallas_call(
        _router_block,
        grid=(g_dim, nt),
        in_specs=[
            pl.BlockSpec((1, _TB, d_dim), lambda g, t: (g, t, 0)),
            pl.BlockSpec((d_dim, e_dim), lambda g, t: (0, 0)),
            pl.BlockSpec((e_dim, 1), lambda g, t: (0, 0)),
            pl.BlockSpec(memory_space=pltpu.SMEM),
            pl.BlockSpec(memory_space=pltpu.SMEM),
        ],
        out_specs=pl.BlockSpec(
            (1, e_dim, _CDIM, _TB), lambda g, t: (g, 0, 0, t)
        ),
        out_shape=jax.ShapeDtypeStruct(
            (g_dim, e_dim, _CDIM, t_dim), jnp.float32
        ),
        scratch_shapes=[pltpu.VMEM((e_dim, 128), jnp.float32)],
        compiler_params=pltpu.CompilerParams(
            dimension_semantics=("arbitrary", "arbitrary"),
        ),
    )(token_inputs, W, b2, ne, cap)
    result = jnp.transpose(out, (0, 3, 1, 2))
    result, _ = jax.lax.optimization_barrier((result, _sc_zeros()))
    return result


# speedup vs baseline: 1.0416x; 1.0416x over previous
import jax
import jax.numpy as jnp
from jax.experimental import pallas as pl
from jax.experimental.pallas import tpu as pltpu

_TB = 1024
_CDIM = 64


def _router_block(x_ref, w_ref, b_ref, ne_ref, cap_ref, out_ref, carry_ref):
    tb = pl.program_id(1)
    e_dim = w_ref.shape[1]

    x = x_ref[0].astype(jnp.bfloat16)
    w = w_ref[...].astype(jnp.bfloat16)
    acc = jnp.dot(x, w, preferred_element_type=jnp.float32)
    acc_t = acc.T
    logits = (acc_t.astype(jnp.bfloat16) + b_ref[...].astype(jnp.bfloat16))
    logits = logits.astype(jnp.float32)

    lmax = jnp.max(logits, axis=0, keepdims=True)
    ex = jnp.exp(logits - lmax)
    ssum = jnp.sum(ex, axis=0, keepdims=True)
    probs = ex / ssum
    pmax = jnp.max(probs, axis=0, keepdims=True)
    eiota = jax.lax.broadcasted_iota(jnp.int32, (e_dim, _TB), 0)
    idx = jnp.min(jnp.where(probs == pmax, eiota, e_dim), axis=0, keepdims=True)

    ne = ne_ref[0, 0]
    mask = ((eiota == idx) & (eiota < ne)).astype(jnp.float32)

    ui = jax.lax.broadcasted_iota(jnp.int32, (_TB, _TB), 0)
    uj = jax.lax.broadcasted_iota(jnp.int32, (_TB, _TB), 1)
    triu = (ui <= uj).astype(jnp.float32)
    cs = jnp.dot(mask, triu, preferred_element_type=jnp.float32)

    @pl.when(tb == 0)
    def _():
        carry_ref[...] = jnp.zeros_like(carry_ref)

    carry = carry_ref[:, 0:1]
    pie = (cs + carry) * mask
    carry_ref[:, 0:1] = carry + cs[:, _TB - 1 : _TB]

    pos = jnp.sum(pie, axis=0, keepdims=True)
    cap = cap_ref[0, 0].astype(jnp.float32)
    wc = (pos > 0.0) & (pos <= cap)
    gate = jnp.where(wc, pmax, 0.0)
    c0 = jnp.where(wc, pos - 1.0, 0.0).astype(jnp.int32)

    gm = mask * gate
    ciota = jax.lax.broadcasted_iota(jnp.int32, (_CDIM, _TB), 0)
    cm = (ciota == c0).astype(jnp.float32)
    out_ref[0] = gm[:, None, :] * cm[None, :, :]


def kernel(token_inputs, W, b, num_experts, expert_capacity):
    g_dim, t_dim, d_dim = token_inputs.shape
    e_dim = W.shape[1]
    nt = t_dim // _TB

    b2 = b.reshape(e_dim, 1)
    ne = jnp.asarray(num_experts, jnp.int32).reshape(1, 1)
    cap = jnp.asarray(expert_capacity, jnp.int32).reshape(1, 1)

    out = pl.pallas_call(
        _router_block,
        grid=(g_dim, nt),
        in_specs=[
            pl.BlockSpec((1, _TB, d_dim), lambda g, t: (g, t, 0)),
            pl.BlockSpec((d_dim, e_dim), lambda g, t: (0, 0)),
            pl.BlockSpec((e_dim, 1), lambda g, t: (0, 0)),
            pl.BlockSpec(memory_space=pltpu.SMEM),
            pl.BlockSpec(memory_space=pltpu.SMEM),
        ],
        out_specs=pl.BlockSpec(
            (1, e_dim, _CDIM, _TB), lambda g, t: (g, 0, 0, t)
        ),
        out_shape=jax.ShapeDtypeStruct(
            (g_dim, e_dim, _CDIM, t_dim), jnp.float32
        ),
        scratch_shapes=[pltpu.VMEM((e_dim, 128), jnp.float32)],
        compiler_params=pltpu.CompilerParams(
            dimension_semantics=("arbitrary", "arbitrary"),
        ),
    )(token_inputs, W, b2, ne, cap)
    return jnp.transpose(out, (0, 3, 1, 2))
